# baseline (speedup 1.0000x reference)
import jax
import jax.numpy as jnp
from jax import lax
from jax.experimental import pallas as pl
from jax.experimental.pallas import tpu as pltpu

_GRID = 20
_EDGE_BLOCK = 8000
_NODE_BLOCK = 1000
_NSLOT = 2


def _mlp3(x, w1_ref, b1_ref, w2_ref, b2_ref, w3_ref, b3_ref):
    h = jnp.dot(x, w1_ref[...], preferred_element_type=jnp.float32) + b1_ref[...]
    h = jnp.maximum(h, 0.0).astype(jnp.bfloat16)
    h = jnp.dot(h, w2_ref[...], preferred_element_type=jnp.float32) + b2_ref[...]
    h = jnp.maximum(h, 0.0).astype(jnp.bfloat16)
    return jnp.dot(h, w3_ref[...], preferred_element_type=jnp.float32) + b3_ref[...]


def _graph_indep_kernel(
    ex_ref, ew1, eb1, ew2, eb2, ew3, eb3,
    nx_ref, nw1, nb1, nw2, nb2, nw3, nb3,
    gx_ref, gw1, gb1, gw2, gb2, gw3, gb3,
    eo_ref, no_ref, go_ref,
    scratch, sem,
):
    i = pl.program_id(0)
    slot = lax.rem(i, _NSLOT)

    @pl.when(i >= _NSLOT)
    def _():
        pltpu.make_async_copy(
            scratch.at[slot],
            eo_ref.at[pl.ds((i - _NSLOT) * _EDGE_BLOCK, _EDGE_BLOCK), :],
            sem.at[slot],
        ).wait()

    scratch[slot] = _mlp3(ex_ref[...], ew1, eb1, ew2, eb2, ew3, eb3)
    pltpu.make_async_copy(
        scratch.at[slot],
        eo_ref.at[pl.ds(i * _EDGE_BLOCK, _EDGE_BLOCK), :],
        sem.at[slot],
    ).start()

    @pl.when(lax.rem(i, 2) == 0)
    def _():
        no_ref[...] = _mlp3(nx_ref[...], nw1, nb1, nw2, nb2, nw3, nb3)

    @pl.when(i == 0)
    def _():
        go_ref[...] = _mlp3(gx_ref[...], gw1, gb1, gw2, gb2, gw3, gb3)

    @pl.when(i == _GRID - 1)
    def _():
        other = lax.rem(i + 1, _NSLOT)
        pltpu.make_async_copy(
            scratch.at[other],
            eo_ref.at[pl.ds((i - 1) * _EDGE_BLOCK, _EDGE_BLOCK), :],
            sem.at[other],
        ).wait()
        pltpu.make_async_copy(
            scratch.at[slot],
            eo_ref.at[pl.ds(i * _EDGE_BLOCK, _EDGE_BLOCK), :],
            sem.at[slot],
        ).wait()


def _prep(x, params):
    w1, b1, w2, b2, w3, b3 = params
    return (
        x.astype(jnp.bfloat16),
        w1.astype(jnp.bfloat16), b1.reshape(1, -1),
        w2.astype(jnp.bfloat16), b2.reshape(1, -1),
        w3.astype(jnp.bfloat16), b3.reshape(1, -1),
    )


@jax.jit
def kernel(nodes, edges, global_attr, node_params, edge_params, global_params):
    n_rows = nodes.shape[0]
    e_rows = edges.shape[0]
    d_out = node_params[-1].shape[0]
    assert e_rows == _GRID * _EDGE_BLOCK and n_rows == (_GRID // 2) * _NODE_BLOCK

    g = jnp.pad(global_attr, ((0, 7), (0, 0)))

    eargs = _prep(edges, edge_params)
    nargs = _prep(nodes, node_params)
    gargs = _prep(g, global_params)

    whole = lambda a: pl.BlockSpec(a.shape, lambda i: (0,) * a.ndim)
    espec = [pl.BlockSpec((_EDGE_BLOCK, edges.shape[1]), lambda i: (i, 0))]
    espec += [whole(a) for a in eargs[1:]]
    nspec = [pl.BlockSpec((_NODE_BLOCK, nodes.shape[1]), lambda i: (i // 2, 0))]
    nspec += [whole(a) for a in nargs[1:]]
    gspec = [whole(a) for a in gargs]

    new_edges, new_nodes, new_global = pl.pallas_call(
        _graph_indep_kernel,
        grid=(_GRID,),
        in_specs=espec + nspec + gspec,
        out_specs=[
            pl.BlockSpec(memory_space=pl.ANY),
            pl.BlockSpec((_NODE_BLOCK, d_out), lambda i: (i // 2, 0)),
            pl.BlockSpec((8, d_out), lambda i: (0, 0)),
        ],
        out_shape=[
            jax.ShapeDtypeStruct((e_rows, d_out), jnp.float32),
            jax.ShapeDtypeStruct((n_rows, d_out), jnp.float32),
            jax.ShapeDtypeStruct((8, d_out), jnp.float32),
        ],
        scratch_shapes=[
            pltpu.VMEM((_NSLOT, _EDGE_BLOCK, d_out), jnp.float32),
            pltpu.SemaphoreType.DMA((_NSLOT,)),
        ],
        compiler_params=pltpu.CompilerParams(
            dimension_semantics=("arbitrary",),
        ),
    )(*eargs, *nargs, *gargs)
    return (new_nodes, new_edges, new_global[:1])


# speedup vs baseline: 1.0001x; 1.0001x over previous
import jax
import jax.numpy as jnp
from jax import lax
from jax.experimental import pallas as pl
from jax.experimental.pallas import tpu as pltpu

_GRID = 20
_EDGE_BLOCK = 8000
_NODE_BLOCK = 1000
_NSLOT = 4


def _mlp3(x, w1_ref, b1_ref, w2_ref, b2_ref, w3_ref, b3_ref):
    h = jnp.dot(x, w1_ref[...], preferred_element_type=jnp.float32) + b1_ref[...]
    h = jnp.maximum(h, 0.0).astype(jnp.bfloat16)
    h = jnp.dot(h, w2_ref[...], preferred_element_type=jnp.float32) + b2_ref[...]
    h = jnp.maximum(h, 0.0).astype(jnp.bfloat16)
    return jnp.dot(h, w3_ref[...], preferred_element_type=jnp.float32) + b3_ref[...]


def _graph_indep_kernel(
    ex_ref, ew1, eb1, ew2, eb2, ew3, eb3,
    nx_ref, nw1, nb1, nw2, nb2, nw3, nb3,
    gx_ref, gw1, gb1, gw2, gb2, gw3, gb3,
    eo_ref, no_ref, go_ref,
    scratch, sem,
):
    i = pl.program_id(0)
    slot = lax.rem(i, _NSLOT)

    @pl.when(i >= _NSLOT)
    def _():
        pltpu.make_async_copy(
            scratch.at[slot],
            eo_ref.at[pl.ds((i - _NSLOT) * _EDGE_BLOCK, _EDGE_BLOCK), :],
            sem.at[slot],
        ).wait()

    scratch[slot] = _mlp3(ex_ref[...], ew1, eb1, ew2, eb2, ew3, eb3)
    pltpu.make_async_copy(
        scratch.at[slot],
        eo_ref.at[pl.ds(i * _EDGE_BLOCK, _EDGE_BLOCK), :],
        sem.at[slot],
    ).start()

    @pl.when(lax.rem(i, 2) == 0)
    def _():
        no_ref[...] = _mlp3(nx_ref[...], nw1, nb1, nw2, nb2, nw3, nb3)

    @pl.when(i == 0)
    def _():
        go_ref[...] = _mlp3(gx_ref[...], gw1, gb1, gw2, gb2, gw3, gb3)

    @pl.when(i == _GRID - 1)
    def _():
        for k in range(_NSLOT - 1, -1, -1):
            s = lax.rem(i - k + _NSLOT, _NSLOT)
            pltpu.make_async_copy(
                scratch.at[s],
                eo_ref.at[pl.ds((i - k) * _EDGE_BLOCK, _EDGE_BLOCK), :],
                sem.at[s],
            ).wait()


def _prep(x, params):
    w1, b1, w2, b2, w3, b3 = params
    return (
        x.astype(jnp.bfloat16),
        w1.astype(jnp.bfloat16), b1.reshape(1, -1),
        w2.astype(jnp.bfloat16), b2.reshape(1, -1),
        w3.astype(jnp.bfloat16), b3.reshape(1, -1),
    )


@jax.jit
def kernel(nodes, edges, global_attr, node_params, edge_params, global_params):
    n_rows = nodes.shape[0]
    e_rows = edges.shape[0]
    d_out = node_params[-1].shape[0]
    assert e_rows == _GRID * _EDGE_BLOCK and n_rows == (_GRID // 2) * _NODE_BLOCK

    g = jnp.pad(global_attr, ((0, 7), (0, 0)))

    eargs = _prep(edges, edge_params)
    nargs = _prep(nodes, node_params)
    gargs = _prep(g, global_params)

    whole = lambda a: pl.BlockSpec(a.shape, lambda i: (0,) * a.ndim)
    espec = [pl.BlockSpec((_EDGE_BLOCK, edges.shape[1]), lambda i: (i, 0))]
    espec += [whole(a) for a in eargs[1:]]
    nspec = [pl.BlockSpec((_NODE_BLOCK, nodes.shape[1]), lambda i: (i // 2, 0))]
    nspec += [whole(a) for a in nargs[1:]]
    gspec = [whole(a) for a in gargs]

    new_edges, new_nodes, new_global = pl.pallas_call(
        _graph_indep_kernel,
        grid=(_GRID,),
        in_specs=espec + nspec + gspec,
        out_specs=[
            pl.BlockSpec(memory_space=pl.ANY),
            pl.BlockSpec((_NODE_BLOCK, d_out), lambda i: (i // 2, 0)),
            pl.BlockSpec((8, d_out), lambda i: (0, 0)),
        ],
        out_shape=[
            jax.ShapeDtypeStruct((e_rows, d_out), jnp.float32),
            jax.ShapeDtypeStruct((n_rows, d_out), jnp.float32),
            jax.ShapeDtypeStruct((8, d_out), jnp.float32),
        ],
        scratch_shapes=[
            pltpu.VMEM((_NSLOT, _EDGE_BLOCK, d_out), jnp.float32),
            pltpu.SemaphoreType.DMA((_NSLOT,)),
        ],
        compiler_params=pltpu.CompilerParams(
            dimension_semantics=("arbitrary",),
        ),
    )(*eargs, *nargs, *gargs)
    return (new_nodes, new_edges, new_global[:1])
